# Initial kernel scaffold; baseline (speedup 1.0000x reference)
import jax
import jax.numpy as jnp
from jax.experimental import pallas as pl


def kernel(edges_data, nodes_data, global_data, receivers, senders):
    raise NotImplementedError("write your pallas kernel here")


# speedup vs baseline: 1.4492x; 1.4492x over previous
import jax
import jax.numpy as jnp
from jax import lax
from jax.experimental import pallas as pl
from jax.experimental.pallas import tpu as pltpu
from jax.experimental.pallas import tpu_sc as plsc

N_NODES = 10000
N_EDGES = 320000
D_EDGE = 16
D_FEAT = 128
D_GLOBAL = 128
D_OUT = D_EDGE + 2 * D_FEAT + D_GLOBAL
C_RECV = D_EDGE
C_SEND = D_EDGE + D_FEAT
C_GLOB = D_EDGE + 2 * D_FEAT

NC = 2
NS = 16
NW = NC * NS
E_PER_W = N_EDGES // NW
B = 80
NCHUNK = E_PER_W // B


def _edge_block(edges_hbm, gtile_hbm, nodes_hbm, recv_hbm, send_hbm, out_hbm,
                idx_r, idx_s, rows_r, rows_s, edge_v, gbuf,
                sem_gr, sem_gs, sem_ge, sem_gb, sem_w1, sem_w2, sem_w3):
    wid = lax.axis_index("s") * NC + lax.axis_index("c")
    wbase = wid * E_PER_W

    pltpu.sync_copy(recv_hbm.at[wid], idx_r)
    pltpu.sync_copy(send_hbm.at[wid], idx_s)
    pltpu.sync_copy(gtile_hbm, gbuf)

    def chunk(c, carry):
        base = wbase + c * B
        cp_e = pltpu.async_copy(edges_hbm.at[pl.ds(base, B)], edge_v, sem_ge)
        cp_g = pltpu.async_copy(
            gbuf, out_hbm.at[pl.ds(base, B), pl.ds(C_GLOB, D_GLOBAL)], sem_gb)
        g_r = pltpu.async_copy(nodes_hbm.at[idx_r.at[c]], rows_r, sem_gr)
        g_s = pltpu.async_copy(nodes_hbm.at[idx_s.at[c]], rows_s, sem_gs)
        cp_e.wait()
        w1 = pltpu.async_copy(
            edge_v, out_hbm.at[pl.ds(base, B), pl.ds(0, D_EDGE)], sem_w1)
        g_r.wait()
        w2 = pltpu.async_copy(
            rows_r, out_hbm.at[pl.ds(base, B), pl.ds(C_RECV, D_FEAT)], sem_w2)
        g_s.wait()
        w3 = pltpu.async_copy(
            rows_s, out_hbm.at[pl.ds(base, B), pl.ds(C_SEND, D_FEAT)], sem_w3)
        w1.wait()
        w2.wait()
        w3.wait()
        cp_g.wait()
        return carry

    lax.fori_loop(0, NCHUNK, chunk, 0)


@jax.jit
def _run(edges_data, g_tile, nodes_data, receivers, senders):
    mesh = plsc.VectorSubcoreMesh(core_axis_name="c", subcore_axis_name="s")
    return pl.kernel(
        _edge_block,
        mesh=mesh,
        compiler_params=pltpu.CompilerParams(use_tc_tiling_on_sc=False),
        out_type=jax.ShapeDtypeStruct((N_EDGES, D_OUT), jnp.float32),
        scratch_types=[
            pltpu.VMEM((NCHUNK, B), jnp.int32),
            pltpu.VMEM((NCHUNK, B), jnp.int32),
            pltpu.VMEM((B, D_FEAT), jnp.float32),
            pltpu.VMEM((B, D_FEAT), jnp.float32),
            pltpu.VMEM((B, D_EDGE), jnp.float32),
            pltpu.VMEM((B, D_GLOBAL), jnp.float32),
            pltpu.SemaphoreType.DMA,
            pltpu.SemaphoreType.DMA,
            pltpu.SemaphoreType.DMA,
            pltpu.SemaphoreType.DMA,
            pltpu.SemaphoreType.DMA,
            pltpu.SemaphoreType.DMA,
            pltpu.SemaphoreType.DMA,
        ],
    )(edges_data, g_tile, nodes_data, receivers, senders)


def kernel(edges_data, nodes_data, global_data, receivers, senders):
    g_tile = jnp.broadcast_to(global_data[None, :], (B, D_GLOBAL))
    recv = receivers.astype(jnp.int32).reshape(NW, NCHUNK, B)
    send = senders.astype(jnp.int32).reshape(NW, NCHUNK, B)
    return _run(edges_data, g_tile, nodes_data, recv, send)


# speedup vs baseline: 1.4542x; 1.0035x over previous
import jax
import jax.numpy as jnp
from jax import lax
from jax.experimental import pallas as pl
from jax.experimental.pallas import tpu as pltpu
from jax.experimental.pallas import tpu_sc as plsc

N_NODES = 10000
N_EDGES = 320000
D_EDGE = 16
D_FEAT = 128
D_GLOBAL = 128
D_OUT = D_EDGE + 2 * D_FEAT + D_GLOBAL
C_RECV = D_EDGE
C_SEND = D_EDGE + D_FEAT
C_GLOB = D_EDGE + 2 * D_FEAT

NC = 2
NS = 16
NW = NC * NS
E_PER_W = N_EDGES // NW
B = 40
NCHUNK = E_PER_W // B
NBUF = 5
GROUPS = NCHUNK // NBUF
K = 2


def _edge_block(edges_hbm, gtile_hbm, nodes_hbm, recv_hbm, send_hbm, out_hbm,
                idx_r, idx_s, rows_r, rows_s, edge_v, gbuf,
                sem_gr, sem_gs, sem_ge, sem_out):
    wid = lax.axis_index("s") * NC + lax.axis_index("c")
    wbase = wid * E_PER_W

    pltpu.sync_copy(recv_hbm.at[wid], idx_r)
    pltpu.sync_copy(send_hbm.at[wid], idx_s)
    pltpu.sync_copy(gtile_hbm, gbuf)

    def issue_inputs(c, b):
        base = wbase + c * B
        pltpu.async_copy(nodes_hbm.at[idx_r.at[c]], rows_r.at[b], sem_gr.at[b])
        pltpu.async_copy(nodes_hbm.at[idx_s.at[c]], rows_s.at[b], sem_gs.at[b])
        pltpu.async_copy(edges_hbm.at[pl.ds(base, B)], edge_v.at[b],
                         sem_ge.at[b])

    def wait_inputs(b):
        pltpu.make_async_copy(nodes_hbm.at[idx_r.at[0]], rows_r.at[b],
                              sem_gr.at[b]).wait()
        pltpu.make_async_copy(nodes_hbm.at[idx_s.at[0]], rows_s.at[b],
                              sem_gs.at[b]).wait()
        pltpu.make_async_copy(edges_hbm.at[pl.ds(0, B)], edge_v.at[b],
                              sem_ge.at[b]).wait()

    def issue_outputs(c, b):
        base = wbase + c * B
        pltpu.async_copy(edge_v.at[b],
                         out_hbm.at[pl.ds(base, B), pl.ds(0, D_EDGE)],
                         sem_out.at[b])
        pltpu.async_copy(rows_r.at[b],
                         out_hbm.at[pl.ds(base, B), pl.ds(C_RECV, D_FEAT)],
                         sem_out.at[b])
        pltpu.async_copy(rows_s.at[b],
                         out_hbm.at[pl.ds(base, B), pl.ds(C_SEND, D_FEAT)],
                         sem_out.at[b])
        pltpu.async_copy(gbuf,
                         out_hbm.at[pl.ds(base, B), pl.ds(C_GLOB, D_GLOBAL)],
                         sem_out.at[b])

    def wait_outputs(b):
        pltpu.make_async_copy(edge_v.at[b],
                              out_hbm.at[pl.ds(0, B), pl.ds(0, D_EDGE)],
                              sem_out.at[b]).wait()
        pltpu.make_async_copy(rows_r.at[b],
                              out_hbm.at[pl.ds(0, B), pl.ds(C_RECV, D_FEAT)],
                              sem_out.at[b]).wait()
        pltpu.make_async_copy(rows_s.at[b],
                              out_hbm.at[pl.ds(0, B), pl.ds(C_SEND, D_FEAT)],
                              sem_out.at[b]).wait()
        pltpu.make_async_copy(gbuf,
                              out_hbm.at[pl.ds(0, B), pl.ds(C_GLOB, D_GLOBAL)],
                              sem_out.at[b]).wait()

    for p in range(K):
        issue_inputs(p, p)

    for b in range(NBUF):
        c = b
        wait_inputs(b)
        issue_outputs(c, b)
        p = c + K
        bp = (b + K) % NBUF
        if p >= NBUF:
            wait_outputs(bp)
        issue_inputs(p, bp)

    def group(g, carry):
        for b in range(NBUF):
            c = g * NBUF + b
            wait_inputs(b)
            issue_outputs(c, b)
            bp = (b + K) % NBUF
            wait_outputs(bp)
            issue_inputs(lax.rem(c + K, NCHUNK), bp)
        return carry

    lax.fori_loop(1, GROUPS, group, 0)

    for b in range(K):
        wait_inputs(b)
    for b in range(K, NBUF):
        wait_outputs(b)


@jax.jit
def _run(edges_data, g_tile, nodes_data, receivers, senders):
    mesh = plsc.VectorSubcoreMesh(core_axis_name="c", subcore_axis_name="s")
    return pl.kernel(
        _edge_block,
        mesh=mesh,
        compiler_params=pltpu.CompilerParams(use_tc_tiling_on_sc=False),
        out_type=jax.ShapeDtypeStruct((N_EDGES, D_OUT), jnp.float32),
        scratch_types=[
            pltpu.VMEM((NCHUNK, B), jnp.int32),
            pltpu.VMEM((NCHUNK, B), jnp.int32),
            pltpu.VMEM((NBUF, B, D_FEAT), jnp.float32),
            pltpu.VMEM((NBUF, B, D_FEAT), jnp.float32),
            pltpu.VMEM((NBUF, B, D_EDGE), jnp.float32),
            pltpu.VMEM((B, D_GLOBAL), jnp.float32),
            pltpu.SemaphoreType.DMA((NBUF,)),
            pltpu.SemaphoreType.DMA((NBUF,)),
            pltpu.SemaphoreType.DMA((NBUF,)),
            pltpu.SemaphoreType.DMA((NBUF,)),
        ],
    )(edges_data, g_tile, nodes_data, receivers, senders)


def kernel(edges_data, nodes_data, global_data, receivers, senders):
    g_tile = jnp.broadcast_to(global_data[None, :], (B, D_GLOBAL))
    recv = receivers.astype(jnp.int32).reshape(NW, NCHUNK, B)
    send = senders.astype(jnp.int32).reshape(NW, NCHUNK, B)
    return _run(edges_data, g_tile, nodes_data, recv, send)


# speedup vs baseline: 1.9788x; 1.3608x over previous
import jax
import jax.numpy as jnp
from jax import lax
from jax.experimental import pallas as pl
from jax.experimental.pallas import tpu as pltpu
from jax.experimental.pallas import tpu_sc as plsc

N_NODES = 10000
N_EDGES = 320000
D_EDGE = 16
D_FEAT = 128
D_GLOBAL = 128
D_OUT = D_EDGE + 2 * D_FEAT + D_GLOBAL
C_RECV = D_EDGE
C_SEND = D_EDGE + D_FEAT
C_GLOB = D_EDGE + 2 * D_FEAT
L = 16

NC = 2
NS = 16
NW = NC * NS
E_PER_W = N_EDGES // NW
B = 40
NCHUNK = E_PER_W // B
PAIRS = NCHUNK // 2
IDX_PAD = 10112


def _edge_block(edges_hbm, glob_hbm, nodes_hbm, recv_hbm, send_hbm, out_hbm,
                idx_r, idx_s, rows_r, rows_s, edge_v, gvec, tile,
                sem_gr, sem_gs, sem_ge, sem_out):
    wid = lax.axis_index("s") * NC + lax.axis_index("c")
    wbase = wid * E_PER_W

    pltpu.sync_copy(recv_hbm.at[pl.ds(wid * IDX_PAD, IDX_PAD)], idx_r)
    pltpu.sync_copy(send_hbm.at[pl.ds(wid * IDX_PAD, IDX_PAD)], idx_s)
    pltpu.sync_copy(glob_hbm, gvec)

    def fill_glob(r, carry):
        for b in range(2):
            for k in range(D_GLOBAL // L):
                tile[b, r, pl.ds(C_GLOB + k * L, L)] = gvec[pl.ds(k * L, L)]
        return carry

    lax.fori_loop(0, B, fill_glob, 0)

    def issue_inputs(c, b):
        base = wbase + c * B
        pltpu.async_copy(nodes_hbm.at[idx_r.at[pl.ds(c * B, B)]], rows_r.at[b],
                         sem_gr.at[b])
        pltpu.async_copy(nodes_hbm.at[idx_s.at[pl.ds(c * B, B)]], rows_s.at[b],
                         sem_gs.at[b])
        pltpu.async_copy(edges_hbm.at[pl.ds(base, B)], edge_v.at[b],
                         sem_ge.at[b])

    def wait_inputs(b):
        pltpu.make_async_copy(nodes_hbm.at[idx_r.at[pl.ds(0, B)]], rows_r.at[b],
                              sem_gr.at[b]).wait()
        pltpu.make_async_copy(nodes_hbm.at[idx_s.at[pl.ds(0, B)]], rows_s.at[b],
                              sem_gs.at[b]).wait()
        pltpu.make_async_copy(edges_hbm.at[pl.ds(0, B)], edge_v.at[b],
                              sem_ge.at[b]).wait()

    def issue_output(c, b):
        base = wbase + c * B
        pltpu.async_copy(tile.at[b], out_hbm.at[pl.ds(base, B)], sem_out.at[b])

    def wait_output(b):
        pltpu.make_async_copy(tile.at[b], out_hbm.at[pl.ds(0, B)],
                              sem_out.at[b]).wait()

    def assemble(b):
        def row(r, carry):
            tile[b, r, pl.ds(0, L)] = edge_v[b, r, pl.ds(0, L)]
            for k in range(D_FEAT // L):
                tile[b, r, pl.ds(C_RECV + k * L, L)] = \
                    rows_r[b, r, pl.ds(k * L, L)]
                tile[b, r, pl.ds(C_SEND + k * L, L)] = \
                    rows_s[b, r, pl.ds(k * L, L)]
            return carry

        lax.fori_loop(0, B, row, 0)

    def step(c, b, first):
        issue_inputs(lax.rem(c + 1, NCHUNK), 1 - b)
        if not first:
            wait_output(b)
        wait_inputs(b)
        assemble(b)
        issue_output(c, b)

    issue_inputs(0, 0)
    step(0, 0, True)
    step(1, 1, True)

    def pair(g, carry):
        step(2 * g, 0, False)
        step(2 * g + 1, 1, False)
        return carry

    lax.fori_loop(1, PAIRS, pair, 0)

    wait_inputs(0)
    wait_output(0)
    wait_output(1)


@jax.jit
def _run(edges_data, global_data, nodes_data, receivers, senders):
    mesh = plsc.VectorSubcoreMesh(core_axis_name="c", subcore_axis_name="s")
    return pl.kernel(
        _edge_block,
        mesh=mesh,
        out_type=jax.ShapeDtypeStruct((N_EDGES, D_OUT), jnp.float32),
        scratch_types=[
            pltpu.VMEM((IDX_PAD,), jnp.int32),
            pltpu.VMEM((IDX_PAD,), jnp.int32),
            pltpu.VMEM((2, B, D_FEAT), jnp.float32),
            pltpu.VMEM((2, B, D_FEAT), jnp.float32),
            pltpu.VMEM((2, B, D_EDGE), jnp.float32),
            pltpu.VMEM((D_GLOBAL,), jnp.float32),
            pltpu.VMEM((2, B, D_OUT), jnp.float32),
            pltpu.SemaphoreType.DMA((2,)),
            pltpu.SemaphoreType.DMA((2,)),
            pltpu.SemaphoreType.DMA((2,)),
            pltpu.SemaphoreType.DMA((2,)),
        ],
    )(edges_data, global_data, nodes_data, receivers, senders)


def kernel(edges_data, nodes_data, global_data, receivers, senders):
    pad = IDX_PAD - E_PER_W
    recv = jnp.pad(receivers.astype(jnp.int32).reshape(NW, E_PER_W),
                   ((0, 0), (0, pad))).reshape(NW * IDX_PAD)
    send = jnp.pad(senders.astype(jnp.int32).reshape(NW, E_PER_W),
                   ((0, 0), (0, pad))).reshape(NW * IDX_PAD)
    return _run(edges_data, global_data, nodes_data, recv, send)


# speedup vs baseline: 2.3276x; 1.1762x over previous
import jax
import jax.numpy as jnp
from jax import lax
from jax.experimental import pallas as pl
from jax.experimental.pallas import tpu as pltpu
from jax.experimental.pallas import tpu_sc as plsc

N_NODES = 10000
N_EDGES = 320000
D_EDGE = 16
D_FEAT = 128
D_GLOBAL = 128
D_OUT = D_EDGE + 2 * D_FEAT + D_GLOBAL
C_RECV = D_EDGE
C_SEND = D_EDGE + D_FEAT
C_GLOB = D_EDGE + 2 * D_FEAT
L = 16

NC = 2
NS = 16
NW = NC * NS
E_PER_W = N_EDGES // NW
B = 40
NCHUNK = E_PER_W // B
PAIRS = NCHUNK // 2
IDX_PAD = 10112


def _edge_block(edges_hbm, glob_hbm, nodes_hbm, recv_hbm, send_hbm, out_hbm,
                idx_r, idx_s, rows_r, rows_s, edge_v, gvec, tile,
                sem_gr, sem_gs, sem_ge, sem_out):
    wid = lax.axis_index("s") * NC + lax.axis_index("c")
    wbase = wid * E_PER_W

    pltpu.sync_copy(recv_hbm.at[pl.ds(wid * IDX_PAD, IDX_PAD)], idx_r)
    pltpu.sync_copy(send_hbm.at[pl.ds(wid * IDX_PAD, IDX_PAD)], idx_s)
    pltpu.sync_copy(glob_hbm, gvec)

    def fill_glob(r, carry):
        for b in range(2):
            for k in range(D_GLOBAL // L):
                tile[b, r, pl.ds(C_GLOB + k * L, L)] = gvec[pl.ds(k * L, L)]
        return carry

    lax.fori_loop(0, B, fill_glob, 0)

    def issue_inputs(c, b):
        base = wbase + c * B
        pltpu.async_copy(nodes_hbm.at[idx_r.at[pl.ds(c * B, B)]], rows_r.at[b],
                         sem_gr.at[b])
        pltpu.async_copy(nodes_hbm.at[idx_s.at[pl.ds(c * B, B)]], rows_s.at[b],
                         sem_gs.at[b])
        pltpu.async_copy(edges_hbm.at[pl.ds(base, B)], edge_v.at[b],
                         sem_ge.at[b])

    def wait_inputs(b):
        pltpu.make_async_copy(nodes_hbm.at[idx_r.at[pl.ds(0, B)]], rows_r.at[b],
                              sem_gr.at[b]).wait()
        pltpu.make_async_copy(nodes_hbm.at[idx_s.at[pl.ds(0, B)]], rows_s.at[b],
                              sem_gs.at[b]).wait()
        pltpu.make_async_copy(edges_hbm.at[pl.ds(0, B)], edge_v.at[b],
                              sem_ge.at[b]).wait()

    def issue_output(c, b):
        base = wbase + c * B
        pltpu.async_copy(tile.at[b], out_hbm.at[pl.ds(base, B)], sem_out.at[b])

    def wait_output(b):
        pltpu.make_async_copy(tile.at[b], out_hbm.at[pl.ds(0, B)],
                              sem_out.at[b]).wait()

    def assemble(b):
        @plsc.parallel_loop(0, B, 1, unroll=2)
        def row(r):
            tile[b, r, pl.ds(0, L)] = edge_v[b, r, pl.ds(0, L)]
            for k in range(D_FEAT // L):
                tile[b, r, pl.ds(C_RECV + k * L, L)] = \
                    rows_r[b, r, pl.ds(k * L, L)]
                tile[b, r, pl.ds(C_SEND + k * L, L)] = \
                    rows_s[b, r, pl.ds(k * L, L)]

    def step(c, b, first):
        issue_inputs(lax.rem(c + 1, NCHUNK), 1 - b)
        if not first:
            wait_output(b)
        wait_inputs(b)
        assemble(b)
        issue_output(c, b)

    issue_inputs(0, 0)
    step(0, 0, True)
    step(1, 1, True)

    def pair(g, carry):
        step(2 * g, 0, False)
        step(2 * g + 1, 1, False)
        return carry

    lax.fori_loop(1, PAIRS, pair, 0)

    wait_inputs(0)
    wait_output(0)
    wait_output(1)


@jax.jit
def _run(edges_data, global_data, nodes_data, receivers, senders):
    mesh = plsc.VectorSubcoreMesh(core_axis_name="c", subcore_axis_name="s")
    return pl.kernel(
        _edge_block,
        mesh=mesh,
        out_type=jax.ShapeDtypeStruct((N_EDGES, D_OUT), jnp.float32),
        scratch_types=[
            pltpu.VMEM((IDX_PAD,), jnp.int32),
            pltpu.VMEM((IDX_PAD,), jnp.int32),
            pltpu.VMEM((2, B, D_FEAT), jnp.float32),
            pltpu.VMEM((2, B, D_FEAT), jnp.float32),
            pltpu.VMEM((2, B, D_EDGE), jnp.float32),
            pltpu.VMEM((D_GLOBAL,), jnp.float32),
            pltpu.VMEM((2, B, D_OUT), jnp.float32),
            pltpu.SemaphoreType.DMA((2,)),
            pltpu.SemaphoreType.DMA((2,)),
            pltpu.SemaphoreType.DMA((2,)),
            pltpu.SemaphoreType.DMA((2,)),
        ],
    )(edges_data, global_data, nodes_data, receivers, senders)


def kernel(edges_data, nodes_data, global_data, receivers, senders):
    pad = IDX_PAD - E_PER_W
    recv = jnp.pad(receivers.astype(jnp.int32).reshape(NW, E_PER_W),
                   ((0, 0), (0, pad))).reshape(NW * IDX_PAD)
    send = jnp.pad(senders.astype(jnp.int32).reshape(NW, E_PER_W),
                   ((0, 0), (0, pad))).reshape(NW * IDX_PAD)
    return _run(edges_data, global_data, nodes_data, recv, send)
